# baseline (speedup 1.0000x reference)
import functools

import jax
import jax.numpy as jnp
from jax import lax
from jax.experimental import pallas as pl
from jax.experimental.pallas import tpu as pltpu
from jax.experimental.pallas import tpu_sc as plsc

DIM = 4096
N_EXPERTS = 64
TOPK = 8
N_GROUPS = 8
GROUP_SIZE = N_EXPERTS // N_GROUPS
TOPK_GROUPS = 4
ROUTE_SCALE = 2.5

BLK = 512
NW = 32
GSEL_ROW = 8


def _scores_body(x_ref, w_ref, s_ref):
    x = x_ref[...].astype(jnp.bfloat16)
    w = w_ref[...].astype(jnp.bfloat16)
    logits = lax.dot_general(
        x, w, (((1,), (1,)), ((), ())),
        preferred_element_type=jnp.float32,
    )
    s_ref[...] = jax.nn.sigmoid(logits)


def _iota16():
    return lax.broadcasted_iota(jnp.int32, (16,), 0)


def _routing_body(tpw, scores_hbm, w_hbm, i_hbm, slab, gsel, wslab, islab):
    nc = 2
    wid = lax.axis_index("s") * nc + lax.axis_index("c")
    base = wid * tpw

    pltpu.sync_copy(scores_hbm.at[pl.ds(base * N_EXPERTS, tpw * N_EXPERTS)],
                    slab)

    iota = _iota16()
    lo8 = iota & 7
    half = iota >= 8

    @plsc.parallel_loop(0, tpw // 16, 1, unroll=4)
    def batch_body(b):
        trow = (b * 16 + iota) * N_EXPERTS
        gmax = []
        for g in range(N_GROUPS):
            m = plsc.load_gather(slab, [trow + (g * GROUP_SIZE)])
            for j in range(1, GROUP_SIZE):
                m = jnp.maximum(
                    m, plsc.load_gather(slab, [trow + (g * GROUP_SIZE + j)]))
            gmax.append(m)
        for k in range(TOPK_GROUPS):
            mx = gmax[0]
            gi = jnp.zeros((16,), jnp.int32)
            for g in range(1, N_GROUPS):
                c = gmax[g] > mx
                mx = jnp.where(c, gmax[g], mx)
                gi = jnp.where(c, g, gi)
            plsc.store_scatter(gsel, [(b * 16 + iota) * GSEL_ROW + k], gi)
            for g in range(N_GROUPS):
                gmax[g] = jnp.where(gi == g, -1.0, gmax[g])

    pat01 = half.astype(jnp.int32)
    pat23 = pat01 + 2
    in8 = iota < 8

    @plsc.parallel_loop(0, tpw, 1, unroll=8)
    def tok_body(t):
        ga = plsc.load_gather(gsel, [t * GSEL_ROW + pat01])
        gb = plsc.load_gather(gsel, [t * GSEL_ROW + pat23])
        expa = ga * GROUP_SIZE + lo8
        expb = gb * GROUP_SIZE + lo8
        va = plsc.load_gather(slab, [t * N_EXPERTS + expa])
        vb = plsc.load_gather(slab, [t * N_EXPERTS + expb])
        ska, sva = plsc.sort_key_val(va, expa, descending=True)
        skb, svb = plsc.sort_key_val(vb, expb, descending=True)
        rkb = lax.rev(skb, (0,))
        rvb = lax.rev(svb, (0,))
        c = ska >= rkb
        mk = jnp.where(c, ska, rkb)
        mv = jnp.where(c, sva, rvb)
        fk, fv = plsc.sort_key_val(mk, mv, descending=True)
        w8 = jnp.where(in8, fk, 0.0)
        s = lax.broadcast_in_dim(jnp.sum(w8), (16,), ())
        wout = w8 * ROUTE_SCALE / s
        plsc.store_scatter(wslab, [t * TOPK + lo8], wout, mask=in8)
        plsc.store_scatter(islab, [t * TOPK + lo8], fv, mask=in8)

    pltpu.sync_copy(wslab, w_hbm.at[pl.ds(base * TOPK, tpw * TOPK)])
    pltpu.sync_copy(islab, i_hbm.at[pl.ds(base * TOPK, tpw * TOPK)])


def _score_chunk(x, W, off_blocks, n_blocks):
    return pl.pallas_call(
        _scores_body,
        grid=(n_blocks,),
        in_specs=[
            pl.BlockSpec((BLK, DIM), lambda i: (i + off_blocks, 0)),
            pl.BlockSpec((N_EXPERTS, DIM), lambda i: (0, 0)),
        ],
        out_specs=pl.BlockSpec((BLK, N_EXPERTS), lambda i: (i, 0)),
        out_shape=jax.ShapeDtypeStruct((n_blocks * BLK, N_EXPERTS),
                                       jnp.float32),
    )(x, W)


def _route_chunk(scores, n_tok):
    tpw = n_tok // NW
    mesh = plsc.VectorSubcoreMesh(core_axis_name="c", subcore_axis_name="s")
    w, i = pl.kernel(
        functools.partial(_routing_body, tpw),
        out_type=[
            jax.ShapeDtypeStruct((n_tok * TOPK,), jnp.float32),
            jax.ShapeDtypeStruct((n_tok * TOPK,), jnp.int32),
        ],
        mesh=mesh,
        compiler_params=pltpu.CompilerParams(needs_layout_passes=False),
        scratch_types=[
            pltpu.VMEM((tpw * N_EXPERTS,), jnp.float32),
            pltpu.VMEM((tpw * GSEL_ROW + 16,), jnp.int32),
            pltpu.VMEM((tpw * TOPK,), jnp.float32),
            pltpu.VMEM((tpw * TOPK,), jnp.int32),
        ],
    )(scores.reshape(-1))
    return w.reshape(n_tok, TOPK), i.reshape(n_tok, TOPK)


CHUNKS = 4


@jax.jit
def kernel(x, W):
    n_tok = x.shape[0]
    blocks = n_tok // BLK // CHUNKS
    ws, idxs = [], []
    for c in range(CHUNKS):
        scores_c = _score_chunk(x, W, c * blocks, blocks)
        w_c, i_c = _route_chunk(scores_c, blocks * BLK)
        ws.append(w_c)
        idxs.append(i_c)
    return jnp.concatenate(ws, axis=0), jnp.concatenate(idxs, axis=0)


# speedup vs baseline: 1.1383x; 1.1383x over previous
import functools

import jax
import jax.numpy as jnp
from jax import lax
from jax.experimental import pallas as pl
from jax.experimental.pallas import tpu as pltpu
from jax.experimental.pallas import tpu_sc as plsc

DIM = 4096
N_EXPERTS = 64
TOPK = 8
N_GROUPS = 8
GROUP_SIZE = N_EXPERTS // N_GROUPS
TOPK_GROUPS = 4
ROUTE_SCALE = 2.5

BLK = 512
NW = 32
GSEL_ROW = 8


def _scores_body(x_ref, w_ref, s_ref):
    x = x_ref[...].astype(jnp.bfloat16)
    w = w_ref[...].astype(jnp.bfloat16)
    logits = lax.dot_general(
        x, w, (((1,), (1,)), ((), ())),
        preferred_element_type=jnp.float32,
    )
    s_ref[...] = jax.nn.sigmoid(logits)


def _iota16():
    return lax.broadcasted_iota(jnp.int32, (16,), 0)


def _routing_body(tpw, scores_hbm, w_hbm, i_hbm, slab, gsel, wslab, islab):
    nc = 2
    wid = lax.axis_index("s") * nc + lax.axis_index("c")
    base = wid * tpw

    pltpu.sync_copy(scores_hbm.at[pl.ds(base * N_EXPERTS, tpw * N_EXPERTS)],
                    slab)

    iota = _iota16()
    lo8 = iota & 7
    half = iota >= 8

    @plsc.parallel_loop(0, tpw // 16, 1, unroll=4)
    def batch_body(b):
        trow = (b * 16 + iota) * N_EXPERTS
        gmax = []
        for g in range(N_GROUPS):
            m = plsc.load_gather(slab, [trow + (g * GROUP_SIZE)])
            for j in range(1, GROUP_SIZE):
                m = jnp.maximum(
                    m, plsc.load_gather(slab, [trow + (g * GROUP_SIZE + j)]))
            gmax.append(m)
        for k in range(TOPK_GROUPS):
            mx = gmax[0]
            gi = jnp.zeros((16,), jnp.int32)
            for g in range(1, N_GROUPS):
                c = gmax[g] > mx
                mx = jnp.where(c, gmax[g], mx)
                gi = jnp.where(c, g, gi)
            plsc.store_scatter(gsel, [(b * 16 + iota) * GSEL_ROW + k], gi)
            for g in range(N_GROUPS):
                gmax[g] = jnp.where(gi == g, -1.0, gmax[g])

    pat01 = half.astype(jnp.int32)
    pat23 = pat01 + 2
    in8 = iota < 8

    @plsc.parallel_loop(0, tpw, 1, unroll=8)
    def tok_body(t):
        ga = plsc.load_gather(gsel, [t * GSEL_ROW + pat01])
        gb = plsc.load_gather(gsel, [t * GSEL_ROW + pat23])
        expa = ga * GROUP_SIZE + lo8
        expb = gb * GROUP_SIZE + lo8
        va = plsc.load_gather(slab, [t * N_EXPERTS + expa])
        vb = plsc.load_gather(slab, [t * N_EXPERTS + expb])
        ska, sva = plsc.sort_key_val(va, expa, descending=True)
        skb, svb = plsc.sort_key_val(vb, expb, descending=True)
        rkb = lax.rev(skb, (0,))
        rvb = lax.rev(svb, (0,))
        c = ska >= rkb
        mk = jnp.where(c, ska, rkb)
        mv = jnp.where(c, sva, rvb)
        fk, fv = plsc.sort_key_val(mk, mv, descending=True)
        w8 = jnp.where(in8, fk, 0.0)
        s = lax.broadcast_in_dim(jnp.sum(w8), (16,), ())
        wout = w8 * ROUTE_SCALE / s
        plsc.store_scatter(wslab, [t * TOPK + lo8], wout, mask=in8)
        plsc.store_scatter(islab, [t * TOPK + lo8], fv, mask=in8)

    pltpu.sync_copy(wslab, w_hbm.at[pl.ds(base * TOPK, tpw * TOPK)])
    pltpu.sync_copy(islab, i_hbm.at[pl.ds(base * TOPK, tpw * TOPK)])


def _score_chunk(x, W, off_blocks, n_blocks):
    return pl.pallas_call(
        _scores_body,
        grid=(n_blocks,),
        in_specs=[
            pl.BlockSpec((BLK, DIM), lambda i: (i + off_blocks, 0)),
            pl.BlockSpec((N_EXPERTS, DIM), lambda i: (0, 0)),
        ],
        out_specs=pl.BlockSpec((BLK, N_EXPERTS), lambda i: (i, 0)),
        out_shape=jax.ShapeDtypeStruct((n_blocks * BLK, N_EXPERTS),
                                       jnp.float32),
    )(x, W)


def _route_chunk(scores, n_tok):
    tpw = n_tok // NW
    mesh = plsc.VectorSubcoreMesh(core_axis_name="c", subcore_axis_name="s")
    w, i = pl.kernel(
        functools.partial(_routing_body, tpw),
        out_type=[
            jax.ShapeDtypeStruct((n_tok * TOPK,), jnp.float32),
            jax.ShapeDtypeStruct((n_tok * TOPK,), jnp.int32),
        ],
        mesh=mesh,
        compiler_params=pltpu.CompilerParams(needs_layout_passes=False),
        scratch_types=[
            pltpu.VMEM((tpw * N_EXPERTS,), jnp.float32),
            pltpu.VMEM((tpw * GSEL_ROW + 16,), jnp.int32),
            pltpu.VMEM((tpw * TOPK,), jnp.float32),
            pltpu.VMEM((tpw * TOPK,), jnp.int32),
        ],
    )(scores.reshape(-1))
    return w.reshape(n_tok, TOPK), i.reshape(n_tok, TOPK)


SPLITS = (10, 6)


@jax.jit
def kernel(x, W):
    ws, idxs = [], []
    off = 0
    for blocks in SPLITS:
        scores_c = _score_chunk(x, W, off, blocks)
        w_c, i_c = _route_chunk(scores_c, blocks * BLK)
        ws.append(w_c)
        idxs.append(i_c)
        off += blocks
    return jnp.concatenate(ws, axis=0), jnp.concatenate(idxs, axis=0)


# speedup vs baseline: 1.1519x; 1.0120x over previous
import functools

import jax
import jax.numpy as jnp
from jax import lax
from jax.experimental import pallas as pl
from jax.experimental.pallas import tpu as pltpu
from jax.experimental.pallas import tpu_sc as plsc

DIM = 4096
N_EXPERTS = 64
TOPK = 8
N_GROUPS = 8
GROUP_SIZE = N_EXPERTS // N_GROUPS
TOPK_GROUPS = 4
ROUTE_SCALE = 2.5

BLK = 512
NW = 32
GSEL_ROW = 8


def _scores_body(x_ref, w_ref, s_ref):
    x = x_ref[...].astype(jnp.bfloat16)
    w = w_ref[...].astype(jnp.bfloat16)
    logits = lax.dot_general(
        x, w, (((1,), (1,)), ((), ())),
        preferred_element_type=jnp.float32,
    )
    s_ref[...] = jax.nn.sigmoid(logits)


def _iota16():
    return lax.broadcasted_iota(jnp.int32, (16,), 0)


def _routing_body(tpw, scores_hbm, w_hbm, i_hbm, slab, gsel, wslab, islab):
    nc = 2
    wid = lax.axis_index("s") * nc + lax.axis_index("c")
    base = wid * tpw

    pltpu.sync_copy(scores_hbm.at[pl.ds(base * N_EXPERTS, tpw * N_EXPERTS)],
                    slab)

    iota = _iota16()
    lo8 = iota & 7
    half = iota >= 8

    @plsc.parallel_loop(0, tpw // 16, 1, unroll=4)
    def batch_body(b):
        trow = (b * 16 + iota) * N_EXPERTS
        gmax = []
        for g in range(N_GROUPS):
            m = plsc.load_gather(slab, [trow + (g * GROUP_SIZE)])
            for j in range(1, GROUP_SIZE):
                m = jnp.maximum(
                    m, plsc.load_gather(slab, [trow + (g * GROUP_SIZE + j)]))
            gmax.append(m)
        for k in range(TOPK_GROUPS):
            mx = gmax[0]
            gi = jnp.zeros((16,), jnp.int32)
            for g in range(1, N_GROUPS):
                c = gmax[g] > mx
                mx = jnp.where(c, gmax[g], mx)
                gi = jnp.where(c, g, gi)
            plsc.store_scatter(gsel, [(b * 16 + iota) * GSEL_ROW + k], gi)
            for g in range(N_GROUPS):
                gmax[g] = jnp.where(gi == g, -1.0, gmax[g])

    pat01 = half.astype(jnp.int32)
    pat23 = pat01 + 2
    in8 = iota < 8

    @plsc.parallel_loop(0, tpw, 1, unroll=8)
    def tok_body(t):
        ga = plsc.load_gather(gsel, [t * GSEL_ROW + pat01])
        gb = plsc.load_gather(gsel, [t * GSEL_ROW + pat23])
        expa = ga * GROUP_SIZE + lo8
        expb = gb * GROUP_SIZE + lo8
        va = plsc.load_gather(slab, [t * N_EXPERTS + expa])
        vb = plsc.load_gather(slab, [t * N_EXPERTS + expb])
        ska, sva = plsc.sort_key_val(va, expa, descending=True)
        skb, svb = plsc.sort_key_val(vb, expb, descending=True)
        rkb = lax.rev(skb, (0,))
        rvb = lax.rev(svb, (0,))
        c = ska >= rkb
        mk = jnp.where(c, ska, rkb)
        mv = jnp.where(c, sva, rvb)
        fk, fv = plsc.sort_key_val(mk, mv, descending=True)
        w8 = jnp.where(in8, fk, 0.0)
        s = lax.broadcast_in_dim(jnp.sum(w8), (16,), ())
        wout = w8 * ROUTE_SCALE / s
        plsc.store_scatter(wslab, [t * TOPK + lo8], wout, mask=in8)
        plsc.store_scatter(islab, [t * TOPK + lo8], fv, mask=in8)

    pltpu.sync_copy(wslab, w_hbm.at[pl.ds(base * TOPK, tpw * TOPK)])
    pltpu.sync_copy(islab, i_hbm.at[pl.ds(base * TOPK, tpw * TOPK)])


def _score_chunk(x, W, off_blocks, n_blocks):
    return pl.pallas_call(
        _scores_body,
        grid=(n_blocks,),
        in_specs=[
            pl.BlockSpec((BLK, DIM), lambda i: (i + off_blocks, 0)),
            pl.BlockSpec((N_EXPERTS, DIM), lambda i: (0, 0)),
        ],
        out_specs=pl.BlockSpec((BLK, N_EXPERTS), lambda i: (i, 0)),
        out_shape=jax.ShapeDtypeStruct((n_blocks * BLK, N_EXPERTS),
                                       jnp.float32),
    )(x, W)


def _route_chunk(scores, n_tok):
    tpw = n_tok // NW
    mesh = plsc.VectorSubcoreMesh(core_axis_name="c", subcore_axis_name="s")
    w, i = pl.kernel(
        functools.partial(_routing_body, tpw),
        out_type=[
            jax.ShapeDtypeStruct((n_tok * TOPK,), jnp.float32),
            jax.ShapeDtypeStruct((n_tok * TOPK,), jnp.int32),
        ],
        mesh=mesh,
        compiler_params=pltpu.CompilerParams(needs_layout_passes=False),
        scratch_types=[
            pltpu.VMEM((tpw * N_EXPERTS,), jnp.float32),
            pltpu.VMEM((tpw * GSEL_ROW + 16,), jnp.int32),
            pltpu.VMEM((tpw * TOPK,), jnp.float32),
            pltpu.VMEM((tpw * TOPK,), jnp.int32),
        ],
    )(scores.reshape(-1))
    return w.reshape(n_tok, TOPK), i.reshape(n_tok, TOPK)


SPLITS = (8, 8)


@jax.jit
def kernel(x, W):
    ws, idxs = [], []
    off = 0
    for blocks in SPLITS:
        scores_c = _score_chunk(x, W, off, blocks)
        w_c, i_c = _route_chunk(scores_c, blocks * BLK)
        ws.append(w_c)
        idxs.append(i_c)
        off += blocks
    return jnp.concatenate(ws, axis=0), jnp.concatenate(idxs, axis=0)


# speedup vs baseline: 1.1526x; 1.0006x over previous
import functools

import jax
import jax.numpy as jnp
from jax import lax
from jax.experimental import pallas as pl
from jax.experimental.pallas import tpu as pltpu
from jax.experimental.pallas import tpu_sc as plsc

DIM = 4096
N_EXPERTS = 64
TOPK = 8
N_GROUPS = 8
GROUP_SIZE = N_EXPERTS // N_GROUPS
TOPK_GROUPS = 4
ROUTE_SCALE = 2.5

BLK = 512
NW = 32
GSEL_ROW = 8


def _scores_body(x_ref, w_ref, s_ref):
    x = x_ref[...].astype(jnp.bfloat16)
    w = w_ref[...].astype(jnp.bfloat16)
    logits = lax.dot_general(
        x, w, (((1,), (1,)), ((), ())),
        preferred_element_type=jnp.float32,
    )
    s_ref[...] = jax.nn.sigmoid(logits)


def _iota16():
    return lax.broadcasted_iota(jnp.int32, (16,), 0)


def _routing_body(tpw, scores_hbm, w_hbm, i_hbm, slab, gsel, wslab, islab):
    nc = 2
    wid = lax.axis_index("s") * nc + lax.axis_index("c")
    base = wid * tpw

    pltpu.sync_copy(scores_hbm.at[pl.ds(base * N_EXPERTS, tpw * N_EXPERTS)],
                    slab)

    iota = _iota16()
    lo8 = iota & 7
    half = iota >= 8

    @plsc.parallel_loop(0, tpw // 16, 1, unroll=4)
    def batch_body(b):
        trow = (b * 16 + iota) * N_EXPERTS
        gmax = []
        for g in range(N_GROUPS):
            m = plsc.load_gather(slab, [trow + (g * GROUP_SIZE)])
            for j in range(1, GROUP_SIZE):
                m = jnp.maximum(
                    m, plsc.load_gather(slab, [trow + (g * GROUP_SIZE + j)]))
            gmax.append(m)
        for k in range(TOPK_GROUPS):
            mx = gmax[0]
            gi = jnp.zeros((16,), jnp.int32)
            for g in range(1, N_GROUPS):
                c = gmax[g] > mx
                mx = jnp.where(c, gmax[g], mx)
                gi = jnp.where(c, g, gi)
            plsc.store_scatter(gsel, [(b * 16 + iota) * GSEL_ROW + k], gi)
            for g in range(N_GROUPS):
                gmax[g] = jnp.where(gi == g, -1.0, gmax[g])

    pat01 = half.astype(jnp.int32)
    pat23 = pat01 + 2
    in8 = iota < 8

    @plsc.parallel_loop(0, tpw, 1, unroll=8)
    def tok_body(t):
        ga = plsc.load_gather(gsel, [t * GSEL_ROW + pat01])
        gb = plsc.load_gather(gsel, [t * GSEL_ROW + pat23])
        expa = ga * GROUP_SIZE + lo8
        expb = gb * GROUP_SIZE + lo8
        va = plsc.load_gather(slab, [t * N_EXPERTS + expa])
        vb = plsc.load_gather(slab, [t * N_EXPERTS + expb])
        ska, sva = plsc.sort_key_val(va, expa, descending=True)
        skb, svb = plsc.sort_key_val(vb, expb, descending=True)
        rkb = lax.rev(skb, (0,))
        rvb = lax.rev(svb, (0,))
        c = ska >= rkb
        mk = jnp.where(c, ska, rkb)
        mv = jnp.where(c, sva, rvb)
        fk, fv = plsc.sort_key_val(mk, mv, descending=True)
        w8 = jnp.where(in8, fk, 0.0)
        s = lax.broadcast_in_dim(jnp.sum(w8), (16,), ())
        wout = w8 * ROUTE_SCALE / s
        plsc.store_scatter(wslab, [t * TOPK + lo8], wout, mask=in8)
        plsc.store_scatter(islab, [t * TOPK + lo8], fv, mask=in8)

    pltpu.sync_copy(wslab, w_hbm.at[pl.ds(base * TOPK, tpw * TOPK)])
    pltpu.sync_copy(islab, i_hbm.at[pl.ds(base * TOPK, tpw * TOPK)])


def _score_chunk(x, W, off_blocks, n_blocks):
    return pl.pallas_call(
        _scores_body,
        grid=(n_blocks,),
        in_specs=[
            pl.BlockSpec((BLK, DIM), lambda i: (i + off_blocks, 0)),
            pl.BlockSpec((N_EXPERTS, DIM), lambda i: (0, 0)),
        ],
        out_specs=pl.BlockSpec((BLK, N_EXPERTS), lambda i: (i, 0)),
        out_shape=jax.ShapeDtypeStruct((n_blocks * BLK, N_EXPERTS),
                                       jnp.float32),
    )(x, W)


def _route_chunk(scores, n_tok):
    tpw = n_tok // NW
    mesh = plsc.VectorSubcoreMesh(core_axis_name="c", subcore_axis_name="s")
    w, i = pl.kernel(
        functools.partial(_routing_body, tpw),
        out_type=[
            jax.ShapeDtypeStruct((n_tok * TOPK,), jnp.float32),
            jax.ShapeDtypeStruct((n_tok * TOPK,), jnp.int32),
        ],
        mesh=mesh,
        compiler_params=pltpu.CompilerParams(needs_layout_passes=False),
        scratch_types=[
            pltpu.VMEM((tpw * N_EXPERTS,), jnp.float32),
            pltpu.VMEM((tpw * GSEL_ROW + 16,), jnp.int32),
            pltpu.VMEM((tpw * TOPK,), jnp.float32),
            pltpu.VMEM((tpw * TOPK,), jnp.int32),
        ],
    )(scores.reshape(-1))
    return w.reshape(n_tok, TOPK), i.reshape(n_tok, TOPK)


SPLITS = (12, 4)


@jax.jit
def kernel(x, W):
    ws, idxs = [], []
    off = 0
    for blocks in SPLITS:
        scores_c = _score_chunk(x, W, off, blocks)
        w_c, i_c = _route_chunk(scores_c, blocks * BLK)
        ws.append(w_c)
        idxs.append(i_c)
        off += blocks
    return jnp.concatenate(ws, axis=0), jnp.concatenate(idxs, axis=0)
